# baseline (speedup 1.0000x reference)
import functools

import jax
import jax.numpy as jnp
from jax import lax
from jax.experimental import pallas as pl
from jax.experimental.pallas import tpu as pltpu
from jax.experimental.pallas import tpu_sc as plsc

_B = 4096
_F = 26
_D = 64
_R = _B * _F
_NC = 2
_NS = 16
_NW = _NC * _NS
_CHUNK = 208
_RPW = _R // _NW
_CPW = _RPW // _CHUNK
_NBUF = 4

_mesh = plsc.VectorSubcoreMesh(core_axis_name="c", subcore_axis_name="s")


@functools.partial(
    pl.kernel,
    mesh=_mesh,
    compiler_params=pltpu.CompilerParams(use_tc_tiling_on_sc=False),
    out_type=jax.ShapeDtypeStruct((_R, _D), jnp.float32),
    scratch_types=[
        pltpu.VMEM((_RPW,), jnp.int32),
        pltpu.VMEM((_NBUF, _CHUNK, 128), jnp.float32),
        pltpu.SemaphoreType.DMA,
        pltpu.SemaphoreType.DMA,
    ],
)
def _gather_rows(idx_hbm, table_hbm, out_hbm, idx_v, rows_v, gsem, wsem):
    wid = lax.axis_index("s") * _NC + lax.axis_index("c")
    rbase = pl.multiple_of(wid * _RPW, _RPW)
    pltpu.sync_copy(idx_hbm.at[pl.ds(rbase, _RPW)], idx_v)

    def g_desc(j):
        return pltpu.make_async_copy(
            table_hbm.at[idx_v.at[pl.ds(j * _CHUNK, _CHUNK)]],
            rows_v.at[j % _NBUF], gsem)

    def w_desc(j):
        return pltpu.make_async_copy(
            rows_v.at[j % _NBUF, :, pl.ds(0, _D)],
            out_hbm.at[pl.ds(rbase + j * _CHUNK, _CHUNK)], wsem)

    g_desc(0).start()
    g_desc(1).start()
    for j in range(_CPW):
        g_desc(j).wait()
        w_desc(j).start()
        if j >= 2:
            w_desc(j - 2).wait()
        if j + 2 < _CPW:
            g_desc(j + 2).start()
    w_desc(_CPW - 2).wait()
    w_desc(_CPW - 1).wait()


def kernel(indices, table):
    idx_flat = indices.astype(jnp.int32).reshape(_R)
    t128 = jnp.pad(table, ((0, 0), (0, 128 - _D)))
    out = _gather_rows(idx_flat, t128)
    return out.reshape(_B, _F * _D)


# speedup vs baseline: 1.2222x; 1.2222x over previous
import functools

import jax
import jax.numpy as jnp
from jax import lax
from jax.experimental import pallas as pl
from jax.experimental.pallas import tpu as pltpu
from jax.experimental.pallas import tpu_sc as plsc

_B = 4096
_F = 26
_D = 64
_R = _B * _F
_NC = 2
_NS = 16
_NW = _NC * _NS
_BPW = _B // _NW
_BPC = 8
_CHUNK = _BPC * _F
_RPW = _R // _NW
_CPW = _BPW // _BPC

_mesh = plsc.VectorSubcoreMesh(core_axis_name="c", subcore_axis_name="s")


@functools.partial(
    pl.kernel,
    mesh=_mesh,
    out_type=jax.ShapeDtypeStruct((_B, _F * _D), jnp.float32),
    scratch_types=[
        pltpu.VMEM((_RPW,), jnp.int32),
        pltpu.VMEM((2, _CHUNK, 128), jnp.float32),
        pltpu.VMEM((2, _BPC, _F * _D), jnp.float32),
        pltpu.SemaphoreType.DMA,
        pltpu.SemaphoreType.DMA,
    ],
)
def _gather_rows(idx_hbm, table_hbm, out_hbm, idx_v, rows_v, img_v, gsem, wsem):
    wid = lax.axis_index("s") * _NC + lax.axis_index("c")
    rbase = pl.multiple_of(wid * _RPW, _RPW)
    bbase = pl.multiple_of(wid * _BPW, _BPW)
    pltpu.sync_copy(idx_hbm.at[pl.ds(rbase, _RPW)], idx_v)

    def g_desc(j, p):
        off = j * _CHUNK if isinstance(j, int) else pl.multiple_of(j * _CHUNK, 8)
        return pltpu.make_async_copy(
            table_hbm.at[idx_v.at[pl.ds(off, _CHUNK)]], rows_v.at[p], gsem)

    def w_desc(j, q):
        off = j * _BPC if isinstance(j, int) else pl.multiple_of(j * _BPC, _BPC)
        return pltpu.make_async_copy(
            img_v.at[q], out_hbm.at[pl.ds(bbase + off, _BPC)], wsem)

    def assemble(p, q):
        for b in range(_BPC):
            for f in range(_F):
                r = b * _F + f
                for u in range(_D // 16):
                    img_v[q, b, pl.ds(f * _D + u * 16, 16)] = (
                        rows_v[p, r, pl.ds(u * 16, 16)])

    g_desc(0, 0).start()

    def body(i, carry):
        for s in range(2):
            j = i * 2 + s
            nxt = 1 - s
            @pl.when(j + 1 < _CPW)
            def _():
                g_desc(j + 1, nxt).start()
            g_desc(j, s).wait()
            @pl.when(j >= 2)
            def _():
                w_desc(j - 2, s).wait()
            assemble(s, s)
            w_desc(j, s).start()
        return carry

    lax.fori_loop(0, _CPW // 2, body, 0)
    w_desc(_CPW - 2, 0).wait()
    w_desc(_CPW - 1, 1).wait()


def kernel(indices, table):
    idx_flat = indices.astype(jnp.int32).reshape(_R)
    t128 = jnp.pad(table, ((0, 0), (0, 128 - _D)))
    return _gather_rows(idx_flat, t128)


# speedup vs baseline: 1.4514x; 1.1875x over previous
import functools

import jax
import jax.numpy as jnp
from jax import lax
from jax.experimental import pallas as pl
from jax.experimental.pallas import tpu as pltpu
from jax.experimental.pallas import tpu_sc as plsc

_B = 4096
_F = 26
_D = 64
_R = _B * _F
_NC = 2
_NS = 16
_NW = _NC * _NS
_BPW = _B // _NW
_BPC = 8
_CHUNK = _BPC * _F
_RPW = _R // _NW
_CPW = _BPW // _BPC

_mesh = plsc.VectorSubcoreMesh(core_axis_name="c", subcore_axis_name="s")

_V = 100000
_CB = 4096


def _transpose_body(tt_ref, out_ref):
    out_ref[:, 0:_D] = tt_ref[...].T


_transpose = pl.pallas_call(
    _transpose_body,
    grid=(_V // _CB + 1,),
    in_specs=[pl.BlockSpec((_D, _CB), lambda i: (0, i))],
    out_specs=pl.BlockSpec((_CB, 128), lambda i: (i, 0)),
    out_shape=jax.ShapeDtypeStruct((_V, 128), jnp.float32),
)


@functools.partial(
    pl.kernel,
    mesh=_mesh,
    out_type=jax.ShapeDtypeStruct((_B, _F * _D), jnp.float32),
    scratch_types=[
        pltpu.VMEM((_RPW,), jnp.int32),
        pltpu.VMEM((2, _CHUNK, 128), jnp.float32),
        pltpu.VMEM((2, _BPC, _F * _D), jnp.float32),
        pltpu.SemaphoreType.DMA,
        pltpu.SemaphoreType.DMA,
    ],
)
def _gather_rows(idx_hbm, table_hbm, out_hbm, idx_v, rows_v, img_v, gsem, wsem):
    wid = lax.axis_index("s") * _NC + lax.axis_index("c")
    rbase = pl.multiple_of(wid * _RPW, _RPW)
    bbase = pl.multiple_of(wid * _BPW, _BPW)
    pltpu.sync_copy(idx_hbm.at[pl.ds(rbase, _RPW)], idx_v)

    def g_desc(j, p):
        off = j * _CHUNK if isinstance(j, int) else pl.multiple_of(j * _CHUNK, 8)
        return pltpu.make_async_copy(
            table_hbm.at[idx_v.at[pl.ds(off, _CHUNK)]], rows_v.at[p], gsem)

    def w_desc(j, q):
        off = j * _BPC if isinstance(j, int) else pl.multiple_of(j * _BPC, _BPC)
        return pltpu.make_async_copy(
            img_v.at[q], out_hbm.at[pl.ds(bbase + off, _BPC)], wsem)

    def assemble(p, q):
        for b in range(_BPC):
            for f in range(_F):
                r = b * _F + f
                for u in range(_D // 16):
                    img_v[q, b, pl.ds(f * _D + u * 16, 16)] = (
                        rows_v[p, r, pl.ds(u * 16, 16)])

    g_desc(0, 0).start()

    def body(i, carry):
        for s in range(2):
            j = i * 2 + s
            nxt = 1 - s
            @pl.when(j + 1 < _CPW)
            def _():
                g_desc(j + 1, nxt).start()
            g_desc(j, s).wait()
            @pl.when(j >= 2)
            def _():
                w_desc(j - 2, s).wait()
            assemble(s, s)
            w_desc(j, s).start()
        return carry

    lax.fori_loop(0, _CPW // 2, body, 0)
    w_desc(_CPW - 2, 0).wait()
    w_desc(_CPW - 1, 1).wait()


def kernel(indices, table):
    idx_flat = indices.astype(jnp.int32).reshape(_R)
    t128 = _transpose(table.T)
    return _gather_rows(idx_flat, t128)
